# baseline (speedup 1.0000x reference)
import numpy as np

import jax
import jax.numpy as jnp
from jax.sharding import Mesh, PartitionSpec as P
from jax.experimental import pallas as pl
from jax.experimental.pallas import tpu as pltpu

N = 4096
E = 256
H = 128
K = 16
BR = 256


def _graph_kernel(Zfull_ref, Zloc_ref, W1T_ref, b1_ref, W2T_ref, b2_ref,
                  zhat_ref, anorm_ref, asp_ref, loss_ref):
    i = pl.program_id(0)
    Zb = Zloc_ref[pl.ds(i * BR, BR), :]
    h = jnp.maximum(
        jnp.dot(Zb, W1T_ref[...], preferred_element_type=jnp.float32)
        + b1_ref[...], 0.0)
    A = jnp.abs(
        jnp.dot(h, W2T_ref[...], preferred_element_type=jnp.float32)
        + b2_ref[...])
    mn = jnp.min(A, axis=1, keepdims=True)
    mx = jnp.max(A, axis=1, keepdims=True)
    An = (A - mn) / (mx - mn + 1e-8)
    anorm_ref[...] = An

    neg = jnp.full((BR, 128), -1.0, dtype=jnp.float32)
    M1 = M2 = M3 = M4 = neg
    for c in range(N // 128):
        v = An[:, c * 128:(c + 1) * 128]
        b1_s = jnp.minimum(M1, v)
        M1 = jnp.maximum(M1, v)
        b2_s = jnp.minimum(M2, b1_s)
        M2 = jnp.maximum(M2, b1_s)
        b3_s = jnp.minimum(M3, b2_s)
        M3 = jnp.maximum(M3, b2_s)
        M4 = jnp.maximum(M4, b3_s)
    Mc = jnp.concatenate([M1, M2, M3, M4], axis=1)
    m = mx * 0.0 + 2.0
    for _ in range(K):
        m = jnp.max(jnp.where(Mc < m, Mc, -1.0), axis=1, keepdims=True)
    sel = An >= m
    nsel = jnp.sum(sel.astype(jnp.float32), axis=1)
    exact = jnp.all(nsel == float(K))

    Asp = jnp.where(sel, An, 0.0)
    asp_ref[...] = Asp
    Zh = jnp.dot(Asp, Zfull_ref[...], preferred_element_type=jnp.float32)
    zhat_ref[...] = Zh
    loss_ref[...] = jnp.sum((Zh - Zb) ** 2).reshape(1, 1, 1)

    @pl.when(jnp.logical_not(exact))
    def _fallback():
        col = jax.lax.broadcasted_iota(jnp.int32, (BR, N), 1)
        w2 = An
        for _ in range(K):
            m2 = jnp.max(w2, axis=1, keepdims=True)
            idx = jnp.min(jnp.where(w2 == m2, col, N), axis=1, keepdims=True)
            w2 = jnp.where(col == idx, -1.0, w2)
        Asp2 = jnp.where(w2 < 0.0, An, 0.0)
        asp_ref[...] = Asp2
        Zh2 = jnp.dot(Asp2, Zfull_ref[...], preferred_element_type=jnp.float32)
        zhat_ref[...] = Zh2
        loss_ref[...] = jnp.sum((Zh2 - Zb) ** 2).reshape(1, 1, 1)


def _shard_body(Z_full, Z_loc, W1T, b1r, W2T, b2r):
    nloc = Z_loc.shape[0]
    grid = nloc // BR
    full = lambda i: (0, 0)
    grid_spec = pl.GridSpec(
        grid=(grid,),
        in_specs=[
            pl.BlockSpec((N, E), full),
            pl.BlockSpec((nloc, E), full),
            pl.BlockSpec((E, H), full),
            pl.BlockSpec((1, H), full),
            pl.BlockSpec((H, N), full),
            pl.BlockSpec((1, N), full),
        ],
        out_specs=(
            pl.BlockSpec((BR, E), lambda i: (i, 0)),
            pl.BlockSpec((BR, N), lambda i: (i, 0)),
            pl.BlockSpec((BR, N), lambda i: (i, 0)),
            pl.BlockSpec((1, 1, 1), lambda i: (i, 0, 0)),
        ),
    )
    out_shapes = (
        jax.ShapeDtypeStruct((nloc, E), jnp.float32),
        jax.ShapeDtypeStruct((nloc, N), jnp.float32),
        jax.ShapeDtypeStruct((nloc, N), jnp.float32),
        jax.ShapeDtypeStruct((grid, 1, 1), jnp.float32),
    )
    Zh, An, Asp, lparts = pl.pallas_call(
        _graph_kernel,
        grid_spec=grid_spec,
        out_shape=out_shapes,
        compiler_params=pltpu.CompilerParams(
            dimension_semantics=("arbitrary",),
        ),
    )(Z_full, Z_loc, W1T, b1r, W2T, b2r)
    return Zh, An, Asp, jnp.sum(lparts).reshape(1)


@jax.jit
def kernel(Z_t, W1, b1, W2, b2):
    W1T = W1.T
    W2T = W2.T
    b1r = b1.reshape(1, H)
    b2r = b2.reshape(1, N)

    devs = jax.devices()
    ndev = 2 if len(devs) >= 2 else 1
    mesh = Mesh(np.array(devs[:ndev]), ("x",))
    f = jax.shard_map(
        _shard_body, mesh=mesh,
        in_specs=(P(), P("x", None), P(), P(), P(), P()),
        out_specs=(P("x", None), P("x", None), P("x", None), P("x")),
        check_vma=False,
    )
    Z_hat, A_norm, A_sparse, lparts = f(Z_t, Z_t, W1T, b1r, W2T, b2r)

    L = (jnp.sum(lparts) / (N * E)).reshape(())
    zero = jnp.zeros((), jnp.float32)
    return (Z_hat, A_norm, A_sparse, A_sparse, L, zero, zero, zero)


# speedup vs baseline: 3.2489x; 3.2489x over previous
import jax
import jax.numpy as jnp
from jax.experimental import pallas as pl
from jax.experimental.pallas import tpu as pltpu

N = 4096
E = 256
H = 128
K = 16
BR = 256


def _graph_kernel(Zfull_ref, Zloc_ref, W1T_ref, b1_ref, W2T_ref, b2_ref,
                  zhat_ref, anorm_ref, asp_ref, loss_ref):
    i = pl.program_id(0)
    Zb = Zloc_ref[pl.ds(i * BR, BR), :]
    h = jnp.maximum(
        jnp.dot(Zb, W1T_ref[...], preferred_element_type=jnp.float32)
        + b1_ref[...], 0.0)
    A = jnp.abs(
        jnp.dot(h, W2T_ref[...], preferred_element_type=jnp.float32)
        + b2_ref[...])
    mn = jnp.min(A, axis=1, keepdims=True)
    mx = jnp.max(A, axis=1, keepdims=True)
    An = (A - mn) / (mx - mn + 1e-8)
    anorm_ref[...] = An

    neg = jnp.full((BR, 128), -1.0, dtype=jnp.float32)
    M1 = M2 = M3 = M4 = neg
    for c in range(N // 128):
        v = An[:, c * 128:(c + 1) * 128]
        b1_s = jnp.minimum(M1, v)
        M1 = jnp.maximum(M1, v)
        b2_s = jnp.minimum(M2, b1_s)
        M2 = jnp.maximum(M2, b1_s)
        b3_s = jnp.minimum(M3, b2_s)
        M3 = jnp.maximum(M3, b2_s)
        M4 = jnp.maximum(M4, b3_s)
    Mc = jnp.concatenate([M1, M2, M3, M4], axis=1)
    m = mx * 0.0 + 2.0
    for _ in range(K):
        m = jnp.max(jnp.where(Mc < m, Mc, -1.0), axis=1, keepdims=True)
    sel = An >= m
    nsel = jnp.sum(sel.astype(jnp.float32), axis=1)
    exact = jnp.all(nsel == float(K))

    Asp = jnp.where(sel, An, 0.0)
    asp_ref[...] = Asp
    Zh = jnp.dot(Asp, Zfull_ref[...], preferred_element_type=jnp.float32)
    zhat_ref[...] = Zh
    loss_ref[...] = jnp.sum((Zh - Zb) ** 2).reshape(1, 1, 1)

    @pl.when(jnp.logical_not(exact))
    def _fallback():
        col = jax.lax.broadcasted_iota(jnp.int32, (BR, N), 1)
        w2 = An
        for _ in range(K):
            m2 = jnp.max(w2, axis=1, keepdims=True)
            idx = jnp.min(jnp.where(w2 == m2, col, N), axis=1, keepdims=True)
            w2 = jnp.where(col == idx, -1.0, w2)
        Asp2 = jnp.where(w2 < 0.0, An, 0.0)
        asp_ref[...] = Asp2
        Zh2 = jnp.dot(Asp2, Zfull_ref[...], preferred_element_type=jnp.float32)
        zhat_ref[...] = Zh2
        loss_ref[...] = jnp.sum((Zh2 - Zb) ** 2).reshape(1, 1, 1)


def _shard_body(Z_full, Z_loc, W1T, b1r, W2T, b2r):
    nloc = Z_loc.shape[0]
    grid = nloc // BR
    full = lambda i: (0, 0)
    grid_spec = pl.GridSpec(
        grid=(grid,),
        in_specs=[
            pl.BlockSpec((N, E), full),
            pl.BlockSpec((nloc, E), full),
            pl.BlockSpec((E, H), full),
            pl.BlockSpec((1, H), full),
            pl.BlockSpec((H, N), full),
            pl.BlockSpec((1, N), full),
        ],
        out_specs=(
            pl.BlockSpec((BR, E), lambda i: (i, 0)),
            pl.BlockSpec((BR, N), lambda i: (i, 0)),
            pl.BlockSpec((BR, N), lambda i: (i, 0)),
            pl.BlockSpec((1, 1, 1), lambda i: (i, 0, 0)),
        ),
    )
    out_shapes = (
        jax.ShapeDtypeStruct((nloc, E), jnp.float32),
        jax.ShapeDtypeStruct((nloc, N), jnp.float32),
        jax.ShapeDtypeStruct((nloc, N), jnp.float32),
        jax.ShapeDtypeStruct((grid, 1, 1), jnp.float32),
    )
    Zh, An, Asp, lparts = pl.pallas_call(
        _graph_kernel,
        grid_spec=grid_spec,
        out_shape=out_shapes,
        compiler_params=pltpu.CompilerParams(
            dimension_semantics=("arbitrary",),
        ),
    )(Z_full, Z_loc, W1T, b1r, W2T, b2r)
    return Zh, An, Asp, jnp.sum(lparts).reshape(1)


@jax.jit
def kernel(Z_t, W1, b1, W2, b2):
    W1T = W1.T
    W2T = W2.T
    b1r = b1.reshape(1, H)
    b2r = b2.reshape(1, N)

    Z_hat, A_norm, A_sparse, lparts = _shard_body(
        Z_t, Z_t, W1T, b1r, W2T, b2r)

    L = (jnp.sum(lparts) / (N * E)).reshape(())
    zero = jnp.zeros((), jnp.float32)
    return (Z_hat, A_norm, A_sparse, A_sparse, L, zero, zero, zero)


# speedup vs baseline: 3.2615x; 1.0039x over previous
import jax
import jax.numpy as jnp
from jax.experimental import pallas as pl
from jax.experimental.pallas import tpu as pltpu

N = 4096
E = 256
H = 128
K = 16
BR = 256


def _graph_kernel(Zfull_ref, Zloc_ref, W1T_ref, b1_ref, W2T_ref, b2_ref,
                  zhat_ref, anorm_ref, asp_ref, loss_ref):
    i = pl.program_id(0)
    Zb = Zloc_ref[pl.ds(i * BR, BR), :]
    h = jnp.maximum(
        jnp.dot(Zb, W1T_ref[...], preferred_element_type=jnp.float32)
        + b1_ref[...], 0.0)
    A = jnp.abs(
        jnp.dot(h, W2T_ref[...], preferred_element_type=jnp.float32)
        + b2_ref[...])
    mn = jnp.min(A, axis=1, keepdims=True)
    mx = jnp.max(A, axis=1, keepdims=True)
    An = (A - mn) / (mx - mn + 1e-8)
    anorm_ref[...] = An

    neg = jnp.full((BR, 128), -1.0, dtype=jnp.float32)
    M1 = M2 = M3 = M4 = neg
    for c in range(N // 128):
        v = An[:, c * 128:(c + 1) * 128]
        b1_s = jnp.minimum(M1, v)
        M1 = jnp.maximum(M1, v)
        b2_s = jnp.minimum(M2, b1_s)
        M2 = jnp.maximum(M2, b1_s)
        b3_s = jnp.minimum(M3, b2_s)
        M3 = jnp.maximum(M3, b2_s)
        M4 = jnp.maximum(M4, b3_s)
    Mc = jnp.concatenate([M1, M2, M3, M4], axis=1)
    m = mx * 0.0 + 2.0
    for _ in range(K):
        m = jnp.max(jnp.where(Mc < m, Mc, -1.0), axis=1, keepdims=True)
    sel = An >= m
    nsel = jnp.sum(sel.astype(jnp.float32), axis=1)
    exact = jnp.all(nsel == float(K))

    Asp = jnp.where(sel, An, 0.0)
    asp_ref[...] = Asp
    Zh = jnp.dot(Asp.astype(jnp.bfloat16),
                 Zfull_ref[...].astype(jnp.bfloat16),
                 preferred_element_type=jnp.float32)
    zhat_ref[...] = Zh
    loss_ref[...] = jnp.sum((Zh - Zb) ** 2).reshape(1, 1, 1)

    @pl.when(jnp.logical_not(exact))
    def _fallback():
        col = jax.lax.broadcasted_iota(jnp.int32, (BR, N), 1)
        w2 = An
        for _ in range(K):
            m2 = jnp.max(w2, axis=1, keepdims=True)
            idx = jnp.min(jnp.where(w2 == m2, col, N), axis=1, keepdims=True)
            w2 = jnp.where(col == idx, -1.0, w2)
        Asp2 = jnp.where(w2 < 0.0, An, 0.0)
        asp_ref[...] = Asp2
        Zh2 = jnp.dot(Asp2, Zfull_ref[...], preferred_element_type=jnp.float32)
        zhat_ref[...] = Zh2
        loss_ref[...] = jnp.sum((Zh2 - Zb) ** 2).reshape(1, 1, 1)


def _shard_body(Z_full, Z_loc, W1T, b1r, W2T, b2r):
    nloc = Z_loc.shape[0]
    grid = nloc // BR
    full = lambda i: (0, 0)
    grid_spec = pl.GridSpec(
        grid=(grid,),
        in_specs=[
            pl.BlockSpec((N, E), full),
            pl.BlockSpec((nloc, E), full),
            pl.BlockSpec((E, H), full),
            pl.BlockSpec((1, H), full),
            pl.BlockSpec((H, N), full),
            pl.BlockSpec((1, N), full),
        ],
        out_specs=(
            pl.BlockSpec((BR, E), lambda i: (i, 0)),
            pl.BlockSpec((BR, N), lambda i: (i, 0)),
            pl.BlockSpec((BR, N), lambda i: (i, 0)),
            pl.BlockSpec((1, 1, 1), lambda i: (i, 0, 0)),
        ),
    )
    out_shapes = (
        jax.ShapeDtypeStruct((nloc, E), jnp.float32),
        jax.ShapeDtypeStruct((nloc, N), jnp.float32),
        jax.ShapeDtypeStruct((nloc, N), jnp.float32),
        jax.ShapeDtypeStruct((grid, 1, 1), jnp.float32),
    )
    Zh, An, Asp, lparts = pl.pallas_call(
        _graph_kernel,
        grid_spec=grid_spec,
        out_shape=out_shapes,
        compiler_params=pltpu.CompilerParams(
            dimension_semantics=("arbitrary",),
        ),
    )(Z_full, Z_loc, W1T, b1r, W2T, b2r)
    return Zh, An, Asp, jnp.sum(lparts).reshape(1)


@jax.jit
def kernel(Z_t, W1, b1, W2, b2):
    W1T = W1.T
    W2T = W2.T
    b1r = b1.reshape(1, H)
    b2r = b2.reshape(1, N)

    Z_hat, A_norm, A_sparse, lparts = _shard_body(
        Z_t, Z_t, W1T, b1r, W2T, b2r)

    L = (jnp.sum(lparts) / (N * E)).reshape(())
    zero = jnp.zeros((), jnp.float32)
    return (Z_hat, A_norm, A_sparse, A_sparse, L, zero, zero, zero)
